# baseline (speedup 1.0000x reference)
import functools

import jax
import jax.numpy as jnp
from jax import lax
from jax.experimental import pallas as pl
from jax.experimental.pallas import tpu as pltpu
from jax.experimental.pallas import tpu_sc as plsc

B = 16384
T = 200
S = 21
N = B * T
TR = T // 8
BC = B // 128

NUM_CORES = 2
NUM_SUBCORES = 16
NW = NUM_CORES * NUM_SUBCORES
NQ = BC // NW
NCHUNK = NQ * TR
NBUF = 5

_mesh = plsc.VectorSubcoreMesh(core_axis_name="c", subcore_axis_name="s")

_OUT_BUFS = [pltpu.VMEM((S, 1, 1, 8, 128), jnp.float32) for _ in range(NBUF)]
_X_BUFS = [pltpu.VMEM((1, 1, 8, 128), jnp.float32) for _ in range(2 * NBUF)]
_SEMS = [pltpu.SemaphoreType.DMA for _ in range(3 * NBUF)]


@functools.partial(
    pl.kernel,
    mesh=_mesh,
    out_type=jax.ShapeDtypeStruct((S, TR, BC, 8, 128), jnp.float32),
    scratch_types=_OUT_BUFS + _X_BUFS + _SEMS,
    compiler_params=pltpu.CompilerParams(needs_layout_passes=False),
)
def _two_hot(x_hbm, out_hbm, *scratch):
    bufs = scratch[:NBUF]
    xbufs = scratch[NBUF : 3 * NBUF]
    osems = scratch[3 * NBUF : 4 * NBUF]
    xsems = scratch[4 * NBUF :]
    wid = lax.axis_index("s") * NUM_CORES + lax.axis_index("c")
    bc0 = wid * NQ

    iota = lax.iota(jnp.int32, 16)
    zeros = jnp.zeros((16,), jnp.float32)
    z0 = jnp.zeros((16,), jnp.int32)

    def x_slice(q, tr):
        return x_hbm.at[pl.ds(tr, 1), pl.ds(bc0 + q, 1)]

    def out_slice(q, tr):
        return out_hbm.at[:, pl.ds(tr, 1), pl.ds(bc0 + q, 1)]

    for sub in range(NBUF):
        pltpu.async_copy(x_slice(0, sub), xbufs[sub], xsems[sub])

    def indices(xb, i):
        t8s = lax.shift_right_logical(i, 3)
        c16 = lax.bitwise_and(i, 7) * 16
        xv = xb[0, 0, t8s, pl.ds(c16, 16)]
        xc = jnp.minimum(jnp.maximum(xv, 0.0), 20.0)
        lo = xc.astype(jnp.int32)
        hi = jnp.minimum(lo + 1, S - 1)
        t8v = jnp.full((16,), t8s, jnp.int32)
        return lo, hi, xc, t8v, c16 + iota

    def process(pg, half, q, tr, sub):
        buf = bufs[sub]
        xb = xbufs[sub + NBUF * half]
        xob = xbufs[sub + NBUF * (1 - half)]
        se = sub + NBUF * half

        pltpu.make_async_copy(x_slice(q, tr), xb, xsems[se]).wait()

        def unzero():
            pltpu.make_async_copy(buf, out_slice(q, tr), osems[sub]).wait()

            def zclear(i, c):
                lo, hi, _, t8v, b128 = indices(xob, i)
                plsc.store_scatter(buf, [hi, z0, z0, t8v, b128], zeros)
                plsc.store_scatter(buf, [lo, z0, z0, t8v, b128], zeros)
                return c

            lax.fori_loop(0, 64, zclear, 0)

        if half == 0:
            @pl.when(pg == 0)
            def _fill():
                def zero_k(k, c):
                    for t8 in range(8):
                        for l in range(8):
                            buf[k, 0, 0, t8, pl.ds(l * 16, 16)] = zeros
                    return c

                lax.fori_loop(0, S, zero_k, 0)

            pl.when(pg > 0)(unzero)
        else:
            unzero()

        wrap = tr + NBUF >= TR
        q_pf = jnp.where(wrap, q + 1, q)
        tr_pf = jnp.where(wrap, tr + NBUF - TR, tr + NBUF)

        @pl.when(q_pf < NQ)
        def _prefetch():
            pltpu.async_copy(
                x_slice(q_pf, tr_pf), xob, xsems[sub + NBUF * (1 - half)]
            )

        def compute(i, c):
            lo, hi, xc, t8v, b128 = indices(xb, i)
            frac = xc - lo.astype(jnp.float32)
            plsc.store_scatter(buf, [hi, z0, z0, t8v, b128], frac)
            plsc.store_scatter(buf, [lo, z0, z0, t8v, b128], 1.0 - frac)
            return c

        lax.fori_loop(0, 64, compute, 0)

        pltpu.async_copy(buf, out_slice(q, tr), osems[sub])

    def incr(q, tr):
        last = tr == TR - 1
        return jnp.where(last, q + 1, q), jnp.where(last, 0, tr + 1)

    def pair_body(pg, qt):
        q, tr = qt
        for half in range(2):
            for sub in range(NBUF):
                process(pg, half, q, tr, sub)
                q, tr = incr(q, tr)
        return q, tr

    lax.fori_loop(
        0, NCHUNK // (2 * NBUF), pair_body, (jnp.int32(0), jnp.int32(0))
    )

    for sub in range(NBUF):
        pltpu.make_async_copy(
            bufs[sub], out_hbm.at[:, pl.ds(0, 1), pl.ds(0, 1)], osems[sub]
        ).wait()


def kernel(x):
    xp = x.reshape(BC, 128, TR, 8).transpose(2, 0, 3, 1)
    out5 = _two_hot(xp)
    return out5.transpose(2, 4, 1, 3, 0).reshape(B, T, S)


# speedup vs baseline: 1.7504x; 1.7504x over previous
import functools

import jax
import jax.numpy as jnp
from jax import lax
from jax.experimental import pallas as pl
from jax.experimental.pallas import tpu as pltpu
from jax.experimental.pallas import tpu_sc as plsc

B = 16384
T = 200
S = 21
N = B * T
TR = T // 8
BC = B // 128

NUM_CORES = 2
NUM_SUBCORES = 16
NW = NUM_CORES * NUM_SUBCORES
NQ = BC // NW
NCHUNK = NQ * TR
NBUF = 5

_mesh = plsc.VectorSubcoreMesh(core_axis_name="c", subcore_axis_name="s")

_OUT_BUFS = [pltpu.VMEM((S, 1, 1, 8, 128), jnp.float32) for _ in range(NBUF)]
_X_BUFS = [pltpu.VMEM((1, 1, 8, 128), jnp.float32) for _ in range(NBUF)]
_SEMS = [pltpu.SemaphoreType.DMA for _ in range(2 * NBUF)]


@functools.partial(
    pl.kernel,
    mesh=_mesh,
    out_type=jax.ShapeDtypeStruct((S, TR, BC, 8, 128), jnp.float32),
    scratch_types=_OUT_BUFS + _X_BUFS + _SEMS,
    compiler_params=pltpu.CompilerParams(needs_layout_passes=False),
)
def _two_hot(x_hbm, out_hbm, *scratch):
    bufs = scratch[:NBUF]
    xbufs = scratch[NBUF : 2 * NBUF]
    osems = scratch[2 * NBUF : 3 * NBUF]
    xsems = scratch[3 * NBUF :]
    wid = lax.axis_index("s") * NUM_CORES + lax.axis_index("c")
    bc0 = wid * NQ

    iota = lax.iota(jnp.int32, 16)
    zeros = jnp.zeros((16,), jnp.float32)
    z0 = jnp.zeros((16,), jnp.int32)

    def x_slice(q, tr):
        return x_hbm.at[pl.ds(tr, 1), pl.ds(bc0 + q, 1)]

    def out_slice(q, tr):
        return out_hbm.at[:, pl.ds(tr, 1), pl.ds(bc0 + q, 1)]

    for sub in range(NBUF):
        pltpu.async_copy(x_slice(0, sub), xbufs[sub], xsems[sub])

    def process(cp, q, tr, sub):
        buf, xb = bufs[sub], xbufs[sub]

        pltpu.make_async_copy(x_slice(q, tr), xb, xsems[sub]).wait()

        @pl.when(cp > 0)
        def _drain():
            pltpu.make_async_copy(buf, out_slice(q, tr), osems[sub]).wait()

        def compute(i, c):
            t8s = lax.shift_right_logical(i, 3)
            c16 = lax.bitwise_and(i, 7) * 16
            b128 = c16 + iota
            xv = xb[0, 0, t8s, pl.ds(c16, 16)]
            xc = jnp.minimum(jnp.maximum(xv, 0.0), 20.0)
            lo = xc.astype(jnp.int32)
            frac = xc - lo.astype(jnp.float32)
            hi = jnp.minimum(lo + 1, S - 1)
            t8v = jnp.full((16,), t8s, jnp.int32)
            for k in range(S):
                buf[k, 0, 0, t8s, pl.ds(c16, 16)] = zeros
            plsc.store_scatter(buf, [hi, z0, z0, t8v, b128], frac)
            plsc.store_scatter(buf, [lo, z0, z0, t8v, b128], 1.0 - frac)
            return c

        lax.fori_loop(0, 64, compute, 0)

        pltpu.async_copy(buf, out_slice(q, tr), osems[sub])

        wrap = tr + NBUF >= TR
        q_pf = jnp.where(wrap, q + 1, q)
        tr_pf = jnp.where(wrap, tr + NBUF - TR, tr + NBUF)

        @pl.when(q_pf < NQ)
        def _prefetch():
            pltpu.async_copy(x_slice(q_pf, tr_pf), xb, xsems[sub])

    def incr(q, tr):
        last = tr == TR - 1
        return jnp.where(last, q + 1, q), jnp.where(last, 0, tr + 1)

    def group_body(cp, qt):
        q, tr = qt
        for sub in range(NBUF):
            process(cp, q, tr, sub)
            q, tr = incr(q, tr)
        return q, tr

    lax.fori_loop(0, NCHUNK // NBUF, group_body, (jnp.int32(0), jnp.int32(0)))

    for sub in range(NBUF):
        pltpu.make_async_copy(
            bufs[sub], out_hbm.at[:, pl.ds(0, 1), pl.ds(0, 1)], osems[sub]
        ).wait()


def kernel(x):
    xp = x.reshape(BC, 128, TR, 8).transpose(2, 0, 3, 1)
    out5 = _two_hot(xp)
    return out5.transpose(2, 4, 1, 3, 0).reshape(B, T, S)
